# baseline (speedup 1.0000x reference)
import functools

import jax
import jax.numpy as jnp
from jax import lax
from jax.experimental import pallas as pl
from jax.experimental.pallas import tpu as pltpu
from jax.experimental.pallas import tpu_sc as plsc

NS = 16
NC = 2

T = 512
SUB = 128


def _celu(x):
    alpha = 0.1
    return jnp.where(x > 0, x, alpha * (jnp.exp(x / alpha) - 1.0))


def _route_tc_body(piece_rows, nt_pad, s_ref, d_ref, te_ref):
    iota_c = lax.broadcasted_iota(jnp.int32, (128, 128), 0)
    iota_c2 = lax.broadcasted_iota(jnp.int32, (128, 128), 1)
    tri_incl = (iota_c <= iota_c2).astype(jnp.float32)
    tri_s = {}
    for rows in set(piece_rows):
        iota_r = lax.broadcasted_iota(jnp.int32, (rows, rows), 0)
        iota_r2 = lax.broadcasted_iota(jnp.int32, (rows, rows), 1)
        tri_s[rows] = (iota_r2 < iota_r).astype(jnp.float32)
    row0 = 0
    for hlf, rows in enumerate(piece_rows):
        tri_strict = tri_s[rows]
        s = s_ref[row0:row0 + rows]
        segs = []
        masks = []
        ranks = []
        tots = []
        for e in range(4):
            m = (s == e).astype(jnp.int32)
            mf = m.astype(jnp.float32)
            within_row = jnp.dot(mf, tri_incl,
                                 preferred_element_type=jnp.float32) - mf
            row_tot = jnp.sum(mf, axis=1, keepdims=True)
            row_prefix = jnp.dot(tri_strict, row_tot,
                                 preferred_element_type=jnp.float32)
            masks.append(m)
            ranks.append((within_row + row_prefix).astype(jnp.int32))
            tots.append(jnp.sum(m))
        ts = jnp.int32(0)
        tss = []
        for e in range(4):
            segs.append(ts * T)
            ts = ts + lax.div(tots[e] + jnp.int32(T - 1), jnp.int32(T))
            tss.append(ts)
        d = jnp.zeros_like(s)
        for e in range(4):
            d = d + masks[e] * (segs[e] + ranks[e])
        d_ref[row0:row0 + rows] = d
        t_vec = lax.broadcasted_iota(jnp.int32, (1, nt_pad), 1) - 1
        te = ((t_vec >= tss[0]).astype(jnp.int32)
              + (t_vec >= tss[1]).astype(jnp.int32)
              + (t_vec >= tss[2]).astype(jnp.int32))
        te_ref[:, hlf * nt_pad:(hlf + 1) * nt_pad] = te
        row0 += rows


def _scatter_rows_body(atom_base, n_atoms, aev_hbm, d_hbm, sorted_hbm,
                       idx0, idx1, rows0, rows1,
                       sem_i0, sem_i1, sem_r0, sem_r1, sem_s0, sem_s1):
    wid = lax.axis_index("c") * NS + lax.axis_index("s")
    chunk = n_atoms // (NC * NS)
    nk = chunk // SUB
    idx_v = [idx0, idx1]
    rows_v = [rows0, rows1]
    sem_i = [sem_i0, sem_i1]
    sem_r = [sem_r0, sem_r1]
    sem_s = [sem_s0, sem_s1]

    def start_load(k):
        p = k % 2
        off = atom_base + wid * chunk + k * SUB
        return (pltpu.async_copy(d_hbm.at[pl.ds(off, SUB)], idx_v[p], sem_i[p]),
                pltpu.async_copy(aev_hbm.at[pl.ds(off, SUB)], rows_v[p], sem_r[p]))

    loads = [None] * nk
    scats = [None] * nk
    loads[0] = start_load(0)
    for k in range(nk):
        p = k % 2
        loads[k][0].wait()
        loads[k][1].wait()
        scats[k] = pltpu.async_copy(rows_v[p], sorted_hbm.at[idx_v[p]], sem_s[p])
        if k + 1 < nk:
            if k >= 1:
                scats[k - 1].wait()
            loads[k + 1] = start_load(k + 1)
    if nk >= 2:
        scats[nk - 2].wait()
    scats[nk - 1].wait()


def _gather_out_body(atom_base, n_atoms, y_hbm, d_hbm, out_hbm,
                     idx_v, o_v, sem):
    wid = lax.axis_index("c") * NS + lax.axis_index("s")
    chunk = n_atoms // (NC * NS)
    base = wid * chunk
    for k in range(chunk // SUB):
        pltpu.sync_copy(d_hbm.at[pl.ds(atom_base + base + k * SUB, SUB)],
                        idx_v)
        for g in range(SUB // 16):
            idx_v[pl.ds(g * 16, 16)] = idx_v[pl.ds(g * 16, 16)] + T
        pltpu.async_copy(y_hbm.at[idx_v], o_v.at[pl.ds(k * SUB, SUB)],
                         sem).wait()
    pltpu.sync_copy(o_v, out_hbm.at[pl.ds(base, chunk)])


def _mlp_body(nt, te_ref, x_ref, w1_ref, b1_ref, w2_ref, b2_ref, w3_ref,
              b3_ref, w4_ref, b4_ref, b0_ref, b1c_ref, o_ref, h1_ref):
    del te_ref, nt
    bf = jnp.bfloat16
    cdim = (((1,), (1,)), ((), ()))
    i = pl.program_id(0)
    par = lax.rem(i, 2)

    hp = h1_ref[1 - par]
    x = x_ref[...].astype(bf)

    h1 = _celu(lax.dot_general(x, w1_ref[0], cdim,
                               preferred_element_type=jnp.float32)
               + b1_ref[0])
    h = _celu(lax.dot_general(hp, w2_ref[0], cdim,
                              preferred_element_type=jnp.float32)
              + b2_ref[0])
    h = _celu(lax.dot_general(h.astype(bf), w3_ref[0], cdim,
                              preferred_element_type=jnp.float32)
              + b3_ref[0])
    y = lax.dot_general(w4_ref[0], h.astype(bf), cdim,
                        preferred_element_type=jnp.float32)
    h1_ref[pl.ds(par, 1)] = h1.astype(bf)[None]
    y = y + b4_ref[0, 0, 0]
    o_ref[...] = (b0_ref[0, 0, 0] + b1c_ref[0, 0, 0] * y)[0]


def kernel(species, aev, W1, B1, W2, B2, W3, B3, W4, B4, b0, b1c):
    B, A = species.shape
    D = aev.shape[-1]
    E, H1 = B1.shape
    H2 = B2.shape[1]
    H3 = B3.shape[1]
    N = B * A
    sizes = [N // 4, N // 2, N // 4]
    bases = [0, N // 4, 3 * N // 4]
    nts = [sz // T + E - 1 for sz in sizes]
    NT_PAD = 128

    species2d = species.reshape(N // 128, 128).astype(jnp.int32)
    aev_flat = aev.reshape(N, D)

    d2d, te2d = pl.pallas_call(
        functools.partial(_route_tc_body, [sz // 128 for sz in sizes],
                          NT_PAD),
        out_shape=(jax.ShapeDtypeStruct((N // 128, 128), jnp.int32),
                   jax.ShapeDtypeStruct((1, len(sizes) * NT_PAD), jnp.int32)),
    )(species2d)
    d = d2d.reshape(N)

    mesh2 = plsc.VectorSubcoreMesh(core_axis_name="c", subcore_axis_name="s")

    def make_scatter(atom_base, n_atoms, np_rows):
        return pl.kernel(
            functools.partial(_scatter_rows_body, atom_base, n_atoms),
            out_type=jax.ShapeDtypeStruct((np_rows, D), jnp.float32),
            mesh=mesh2,
            scratch_types=[
                pltpu.VMEM((SUB,), jnp.int32),
                pltpu.VMEM((SUB,), jnp.int32),
                pltpu.VMEM((SUB, D), jnp.float32),
                pltpu.VMEM((SUB, D), jnp.float32),
                pltpu.SemaphoreType.DMA,
                pltpu.SemaphoreType.DMA,
                pltpu.SemaphoreType.DMA,
                pltpu.SemaphoreType.DMA,
                pltpu.SemaphoreType.DMA,
                pltpu.SemaphoreType.DMA,
            ],
        )

    bf = jnp.bfloat16
    W1b = W1.astype(bf)
    W2b = W2.astype(bf)
    W3b = W3.astype(bf)
    W4b = W4.astype(bf)
    B1r = B1.reshape(E, 1, H1)
    B2r = B2.reshape(E, 1, H2)
    B3r = B3.reshape(E, 1, H3)
    B4r = B4.reshape(E, 1, 1)
    b0r = b0.reshape(E, 1, 1)
    b1cr = b1c.reshape(E, 1, 1)

    def run_mlp(piece, nt, np_rows, sorted_aev):
        off = piece * NT_PAD

        def wmap1(i, te_r):
            return (te_r[0, off + i + 1], 0, 0)

        def wmap2(i, te_r):
            return (te_r[0, off + i], 0, 0)

        grid_spec = pltpu.PrefetchScalarGridSpec(
            num_scalar_prefetch=1,
            grid=(nt + 1,),
            in_specs=[
                pl.BlockSpec((T, D), lambda i, te_r: (i, 0)),
                pl.BlockSpec((1, H1, D), wmap1),
                pl.BlockSpec((1, 1, H1), wmap1),
                pl.BlockSpec((1, H2, H1), wmap2),
                pl.BlockSpec((1, 1, H2), wmap2),
                pl.BlockSpec((1, H3, H2), wmap2),
                pl.BlockSpec((1, 1, H3), wmap2),
                pl.BlockSpec((1, 1, H3), wmap2),
                pl.BlockSpec((1, 1, 1), wmap2),
                pl.BlockSpec((1, 1, 1), wmap2),
                pl.BlockSpec((1, 1, 1), wmap2),
            ],
            out_specs=pl.BlockSpec((T,), lambda i, te_r: (i,)),
            scratch_shapes=[pltpu.VMEM((2, T, H1), jnp.bfloat16)],
        )
        return pl.pallas_call(
            functools.partial(_mlp_body, nt),
            grid_spec=grid_spec,
            out_shape=jax.ShapeDtypeStruct((np_rows + T,), jnp.float32),
        )(te2d, sorted_aev, W1b, B1r, W2b, B2r, W3b, B3r, W4b, B4r,
          b0r, b1cr)

    def make_gather(atom_base, n_atoms):
        return pl.kernel(
            functools.partial(_gather_out_body, atom_base, n_atoms),
            out_type=jax.ShapeDtypeStruct((n_atoms,), jnp.float32),
            mesh=mesh2,
            scratch_types=[
                pltpu.VMEM((SUB,), jnp.int32),
                pltpu.VMEM((n_atoms // (NC * NS),), jnp.float32),
                pltpu.SemaphoreType.DMA,
            ],
        )

    sorteds = [make_scatter(bases[i], sizes[i], (nts[i] + 1) * T)(aev_flat, d)
               for i in range(len(sizes))]
    ys = [run_mlp(i, nts[i], nts[i] * T, sorteds[i])
          for i in range(len(sizes))]
    outs = [make_gather(bases[i], sizes[i])(ys[i], d)
            for i in range(len(sizes))]
    return jnp.concatenate(outs).reshape(B, A)


# speedup vs baseline: 1.0040x; 1.0040x over previous
import functools

import jax
import jax.numpy as jnp
from jax import lax
from jax.experimental import pallas as pl
from jax.experimental.pallas import tpu as pltpu
from jax.experimental.pallas import tpu_sc as plsc

NS = 16
NC = 2

T = 512
SUB = 128


def _celu(x):
    alpha = 0.1
    return jnp.where(x > 0, x, alpha * (jnp.exp(x / alpha) - 1.0))


def _route_tc_body(piece_rows, nt_pad, s_ref, d_ref, te_ref):
    iota_c = lax.broadcasted_iota(jnp.int32, (128, 128), 0)
    iota_c2 = lax.broadcasted_iota(jnp.int32, (128, 128), 1)
    tri_incl = (iota_c <= iota_c2).astype(jnp.float32)
    tri_s = {}
    for rows in set(piece_rows):
        iota_r = lax.broadcasted_iota(jnp.int32, (rows, rows), 0)
        iota_r2 = lax.broadcasted_iota(jnp.int32, (rows, rows), 1)
        tri_s[rows] = (iota_r2 < iota_r).astype(jnp.float32)
    row0 = 0
    for hlf, rows in enumerate(piece_rows):
        tri_strict = tri_s[rows]
        s = s_ref[row0:row0 + rows]
        segs = []
        masks = []
        ranks = []
        tots = []
        for e in range(4):
            m = (s == e).astype(jnp.int32)
            mf = m.astype(jnp.float32)
            within_row = jnp.dot(mf, tri_incl,
                                 preferred_element_type=jnp.float32) - mf
            row_tot = jnp.sum(mf, axis=1, keepdims=True)
            row_prefix = jnp.dot(tri_strict, row_tot,
                                 preferred_element_type=jnp.float32)
            masks.append(m)
            ranks.append((within_row + row_prefix).astype(jnp.int32))
            tots.append(jnp.sum(m))
        ts = jnp.int32(0)
        tss = []
        for e in range(4):
            segs.append(ts * T)
            ts = ts + lax.div(tots[e] + jnp.int32(T - 1), jnp.int32(T))
            tss.append(ts)
        d = jnp.zeros_like(s)
        for e in range(4):
            d = d + masks[e] * (segs[e] + ranks[e])
        d_ref[row0:row0 + rows] = d
        t_vec = lax.broadcasted_iota(jnp.int32, (1, nt_pad), 1)
        te = ((t_vec >= tss[0]).astype(jnp.int32)
              + (t_vec >= tss[1]).astype(jnp.int32)
              + (t_vec >= tss[2]).astype(jnp.int32))
        te_ref[:, hlf * nt_pad:(hlf + 1) * nt_pad] = te
        row0 += rows


def _scatter_rows_body(atom_base, n_atoms, aev_hbm, d_hbm, sorted_hbm,
                       idx0, idx1, rows0, rows1,
                       sem_i0, sem_i1, sem_r0, sem_r1, sem_s0, sem_s1):
    wid = lax.axis_index("c") * NS + lax.axis_index("s")
    chunk = n_atoms // (NC * NS)
    nk = chunk // SUB
    idx_v = [idx0, idx1]
    rows_v = [rows0, rows1]
    sem_i = [sem_i0, sem_i1]
    sem_r = [sem_r0, sem_r1]
    sem_s = [sem_s0, sem_s1]

    def start_load(k):
        p = k % 2
        off = atom_base + wid * chunk + k * SUB
        return (pltpu.async_copy(d_hbm.at[pl.ds(off, SUB)], idx_v[p], sem_i[p]),
                pltpu.async_copy(aev_hbm.at[pl.ds(off, SUB)], rows_v[p], sem_r[p]))

    loads = [None] * nk
    scats = [None] * nk
    loads[0] = start_load(0)
    for k in range(nk):
        p = k % 2
        loads[k][0].wait()
        loads[k][1].wait()
        scats[k] = pltpu.async_copy(rows_v[p], sorted_hbm.at[idx_v[p]], sem_s[p])
        if k + 1 < nk:
            if k >= 1:
                scats[k - 1].wait()
            loads[k + 1] = start_load(k + 1)
    if nk >= 2:
        scats[nk - 2].wait()
    scats[nk - 1].wait()


def _gather_out_body(atom_base, n_atoms, y_hbm, d_hbm, out_hbm,
                     idx_v, o_v, sem):
    wid = lax.axis_index("c") * NS + lax.axis_index("s")
    chunk = n_atoms // (NC * NS)
    base = wid * chunk
    for k in range(chunk // SUB):
        pltpu.sync_copy(d_hbm.at[pl.ds(atom_base + base + k * SUB, SUB)],
                        idx_v)
        pltpu.async_copy(y_hbm.at[idx_v], o_v.at[pl.ds(k * SUB, SUB)],
                         sem).wait()
    pltpu.sync_copy(o_v, out_hbm.at[pl.ds(base, chunk)])


def _mlp_body(nt, te_ref, x_ref, w1_ref, b1_ref, w2_ref, b2_ref, w3_ref,
              b3_ref, w4_ref, b4_ref, b0_ref, b1c_ref, o_ref, h1_ref):
    del te_ref, nt
    bf = jnp.bfloat16
    cdim = (((1,), (1,)), ((), ()))
    i = pl.program_id(0)
    par = lax.rem(i, 2)

    hp = h1_ref[1 - par]
    x = x_ref[...].astype(bf)

    h1 = _celu(lax.dot_general(x, w1_ref[0], cdim,
                               preferred_element_type=jnp.float32)
               + b1_ref[0])
    h = _celu(lax.dot_general(hp, w2_ref[0], cdim,
                              preferred_element_type=jnp.float32)
              + b2_ref[0])
    h = _celu(lax.dot_general(h.astype(bf), w3_ref[0], cdim,
                              preferred_element_type=jnp.float32)
              + b3_ref[0])
    y = lax.dot_general(w4_ref[0], h.astype(bf), cdim,
                        preferred_element_type=jnp.float32)
    h1_ref[pl.ds(par, 1)] = h1.astype(bf)[None]
    y = y + b4_ref[0, 0, 0]
    o_ref[...] = (b0_ref[0, 0, 0] + b1c_ref[0, 0, 0] * y)[0]


def kernel(species, aev, W1, B1, W2, B2, W3, B3, W4, B4, b0, b1c):
    B, A = species.shape
    D = aev.shape[-1]
    E, H1 = B1.shape
    H2 = B2.shape[1]
    H3 = B3.shape[1]
    N = B * A
    sizes = [N // 4, N // 2, N // 4]
    bases = [0, N // 4, 3 * N // 4]
    nts = [sz // T + E - 1 for sz in sizes]
    NT_PAD = 128

    species2d = species.reshape(N // 128, 128).astype(jnp.int32)
    aev_flat = aev.reshape(N, D)

    d2d, te2d = pl.pallas_call(
        functools.partial(_route_tc_body, [sz // 128 for sz in sizes],
                          NT_PAD),
        out_shape=(jax.ShapeDtypeStruct((N // 128, 128), jnp.int32),
                   jax.ShapeDtypeStruct((1, len(sizes) * NT_PAD), jnp.int32)),
    )(species2d)
    d = d2d.reshape(N)

    mesh2 = plsc.VectorSubcoreMesh(core_axis_name="c", subcore_axis_name="s")

    def make_scatter(atom_base, n_atoms, np_rows):
        return pl.kernel(
            functools.partial(_scatter_rows_body, atom_base, n_atoms),
            out_type=jax.ShapeDtypeStruct((np_rows, D), jnp.float32),
            mesh=mesh2,
            scratch_types=[
                pltpu.VMEM((SUB,), jnp.int32),
                pltpu.VMEM((SUB,), jnp.int32),
                pltpu.VMEM((SUB, D), jnp.float32),
                pltpu.VMEM((SUB, D), jnp.float32),
                pltpu.SemaphoreType.DMA,
                pltpu.SemaphoreType.DMA,
                pltpu.SemaphoreType.DMA,
                pltpu.SemaphoreType.DMA,
                pltpu.SemaphoreType.DMA,
                pltpu.SemaphoreType.DMA,
            ],
        )

    bf = jnp.bfloat16
    W1b = W1.astype(bf)
    W2b = W2.astype(bf)
    W3b = W3.astype(bf)
    W4b = W4.astype(bf)
    B1r = B1.reshape(E, 1, H1)
    B2r = B2.reshape(E, 1, H2)
    B3r = B3.reshape(E, 1, H3)
    B4r = B4.reshape(E, 1, 1)
    b0r = b0.reshape(E, 1, 1)
    b1cr = b1c.reshape(E, 1, 1)

    def run_mlp(piece, nt, np_rows, sorted_aev):
        off = piece * NT_PAD

        def wmap1(i, te_r):
            return (te_r[0, off + jnp.minimum(i, nt - 1)], 0, 0)

        def wmap2(i, te_r):
            return (te_r[0, off + jnp.maximum(i - 1, 0)], 0, 0)

        grid_spec = pltpu.PrefetchScalarGridSpec(
            num_scalar_prefetch=1,
            grid=(nt + 1,),
            in_specs=[
                pl.BlockSpec((T, D),
                             lambda i, te_r: (jnp.minimum(i, nt - 1), 0)),
                pl.BlockSpec((1, H1, D), wmap1),
                pl.BlockSpec((1, 1, H1), wmap1),
                pl.BlockSpec((1, H2, H1), wmap2),
                pl.BlockSpec((1, 1, H2), wmap2),
                pl.BlockSpec((1, H3, H2), wmap2),
                pl.BlockSpec((1, 1, H3), wmap2),
                pl.BlockSpec((1, 1, H3), wmap2),
                pl.BlockSpec((1, 1, 1), wmap2),
                pl.BlockSpec((1, 1, 1), wmap2),
                pl.BlockSpec((1, 1, 1), wmap2),
            ],
            out_specs=pl.BlockSpec(
                (T,), lambda i, te_r: (jnp.maximum(i - 1, 0),)),
            scratch_shapes=[pltpu.VMEM((2, T, H1), jnp.bfloat16)],
        )
        return pl.pallas_call(
            functools.partial(_mlp_body, nt),
            grid_spec=grid_spec,
            out_shape=jax.ShapeDtypeStruct((np_rows,), jnp.float32),
        )(te2d, sorted_aev, W1b, B1r, W2b, B2r, W3b, B3r, W4b, B4r,
          b0r, b1cr)

    def make_gather(atom_base, n_atoms):
        return pl.kernel(
            functools.partial(_gather_out_body, atom_base, n_atoms),
            out_type=jax.ShapeDtypeStruct((n_atoms,), jnp.float32),
            mesh=mesh2,
            scratch_types=[
                pltpu.VMEM((SUB,), jnp.int32),
                pltpu.VMEM((n_atoms // (NC * NS),), jnp.float32),
                pltpu.SemaphoreType.DMA,
            ],
        )

    sorteds = [make_scatter(bases[i], sizes[i], nts[i] * T)(aev_flat, d)
               for i in range(len(sizes))]
    ys = [run_mlp(i, nts[i], nts[i] * T, sorteds[i])
          for i in range(len(sizes))]
    outs = [make_gather(bases[i], sizes[i])(ys[i], d)
            for i in range(len(sizes))]
    return jnp.concatenate(outs).reshape(B, A)


# speedup vs baseline: 1.0113x; 1.0073x over previous
import functools

import jax
import jax.numpy as jnp
from jax import lax
from jax.experimental import pallas as pl
from jax.experimental.pallas import tpu as pltpu
from jax.experimental.pallas import tpu_sc as plsc

NS = 16
NC = 2

T = 512
SUB = 128


def _celu(x):
    alpha = 0.1
    return jnp.where(x > 0, x, alpha * (jnp.exp(x / alpha) - 1.0))


def _route_tc_body(piece_rows, nt_pad, s_ref, d_ref, te_ref):
    iota_c = lax.broadcasted_iota(jnp.int32, (128, 128), 0)
    iota_c2 = lax.broadcasted_iota(jnp.int32, (128, 128), 1)
    tri_incl = (iota_c <= iota_c2).astype(jnp.float32)
    tri_s = {}
    for rows in set(piece_rows):
        iota_r = lax.broadcasted_iota(jnp.int32, (rows, rows), 0)
        iota_r2 = lax.broadcasted_iota(jnp.int32, (rows, rows), 1)
        tri_s[rows] = (iota_r2 < iota_r).astype(jnp.float32)
    row0 = 0
    for hlf, rows in enumerate(piece_rows):
        tri_strict = tri_s[rows]
        s = s_ref[row0:row0 + rows]
        segs = []
        masks = []
        ranks = []
        tots = []
        for e in range(4):
            m = (s == e).astype(jnp.int32)
            mf = m.astype(jnp.float32)
            within_row = jnp.dot(mf, tri_incl,
                                 preferred_element_type=jnp.float32) - mf
            row_tot = jnp.sum(mf, axis=1, keepdims=True)
            row_prefix = jnp.dot(tri_strict, row_tot,
                                 preferred_element_type=jnp.float32)
            masks.append(m)
            ranks.append((within_row + row_prefix).astype(jnp.int32))
            tots.append(jnp.sum(m))
        ts = jnp.int32(0)
        tss = []
        for e in range(4):
            segs.append(ts * T)
            ts = ts + lax.div(tots[e] + jnp.int32(T - 1), jnp.int32(T))
            tss.append(ts)
        d = jnp.zeros_like(s)
        for e in range(4):
            d = d + masks[e] * (segs[e] + ranks[e])
        d_ref[row0:row0 + rows] = d
        t_vec = lax.broadcasted_iota(jnp.int32, (1, nt_pad), 1)
        te = ((t_vec >= tss[0]).astype(jnp.int32)
              + (t_vec >= tss[1]).astype(jnp.int32)
              + (t_vec >= tss[2]).astype(jnp.int32))
        te_ref[:, hlf * nt_pad:(hlf + 1) * nt_pad] = te
        row0 += rows


def _scatter_rows_body(atom_base, n_atoms, aev_hbm, d_hbm, sorted_hbm,
                       idx0, idx1, rows0, rows1,
                       sem_i0, sem_i1, sem_r0, sem_r1, sem_s0, sem_s1):
    wid = lax.axis_index("c") * NS + lax.axis_index("s")
    chunk = n_atoms // (NC * NS)
    nk = chunk // SUB
    idx_v = [idx0, idx1]
    rows_v = [rows0, rows1]
    sem_i = [sem_i0, sem_i1]
    sem_r = [sem_r0, sem_r1]
    sem_s = [sem_s0, sem_s1]

    def start_load(k):
        p = k % 2
        off = atom_base + wid * chunk + k * SUB
        return (pltpu.async_copy(d_hbm.at[pl.ds(off, SUB)], idx_v[p], sem_i[p]),
                pltpu.async_copy(aev_hbm.at[pl.ds(off, SUB)], rows_v[p], sem_r[p]))

    loads = [None] * nk
    scats = [None] * nk
    loads[0] = start_load(0)
    for k in range(nk):
        p = k % 2
        loads[k][0].wait()
        loads[k][1].wait()
        scats[k] = pltpu.async_copy(rows_v[p], sorted_hbm.at[idx_v[p]], sem_s[p])
        if k + 1 < nk:
            if k >= 1:
                scats[k - 1].wait()
            loads[k + 1] = start_load(k + 1)
    if nk >= 2:
        scats[nk - 2].wait()
    scats[nk - 1].wait()


def _gather_out_body(atom_base, n_atoms, y_hbm, d_hbm, out_hbm,
                     idx0, idx1, o_v, sem_i0, sem_i1, sem_g0, sem_g1):
    wid = lax.axis_index("c") * NS + lax.axis_index("s")
    chunk = n_atoms // (NC * NS)
    nk = chunk // SUB
    base = wid * chunk
    idx_v = [idx0, idx1]
    sem_i = [sem_i0, sem_i1]
    sem_g = [sem_g0, sem_g1]

    def start_idx(k):
        p = k % 2
        return pltpu.async_copy(
            d_hbm.at[pl.ds(atom_base + base + k * SUB, SUB)],
            idx_v[p], sem_i[p])

    il = [None] * nk
    gl = [None] * nk
    il[0] = start_idx(0)
    for k in range(nk):
        p = k % 2
        il[k].wait()
        gl[k] = pltpu.async_copy(y_hbm.at[idx_v[p]],
                                 o_v.at[pl.ds(k * SUB, SUB)], sem_g[p])
        if k + 1 < nk:
            if k >= 1:
                gl[k - 1].wait()
            il[k + 1] = start_idx(k + 1)
    if nk >= 2:
        gl[nk - 2].wait()
    gl[nk - 1].wait()
    pltpu.sync_copy(o_v, out_hbm.at[pl.ds(base, chunk)])


def _mlp_body(nt, te_ref, x_ref, w1_ref, b1_ref, w2_ref, b2_ref, w3_ref,
              b3_ref, w4_ref, b4_ref, b0_ref, b1c_ref, o_ref, h1_ref):
    del te_ref, nt
    bf = jnp.bfloat16
    cdim = (((1,), (1,)), ((), ()))
    i = pl.program_id(0)
    par = lax.rem(i, 2)

    hp = h1_ref[1 - par]
    x = x_ref[...].astype(bf)

    h1 = _celu(lax.dot_general(x, w1_ref[0], cdim,
                               preferred_element_type=jnp.float32)
               + b1_ref[0])
    h = _celu(lax.dot_general(hp, w2_ref[0], cdim,
                              preferred_element_type=jnp.float32)
              + b2_ref[0])
    h = _celu(lax.dot_general(h.astype(bf), w3_ref[0], cdim,
                              preferred_element_type=jnp.float32)
              + b3_ref[0])
    y = lax.dot_general(w4_ref[0], h.astype(bf), cdim,
                        preferred_element_type=jnp.float32)
    h1_ref[pl.ds(par, 1)] = h1.astype(bf)[None]
    y = y + b4_ref[0, 0, 0]
    o_ref[...] = (b0_ref[0, 0, 0] + b1c_ref[0, 0, 0] * y)[0]


def kernel(species, aev, W1, B1, W2, B2, W3, B3, W4, B4, b0, b1c):
    B, A = species.shape
    D = aev.shape[-1]
    E, H1 = B1.shape
    H2 = B2.shape[1]
    H3 = B3.shape[1]
    N = B * A
    sizes = [N // 4, N // 2, N // 4]
    bases = [0, N // 4, 3 * N // 4]
    nts = [sz // T + E - 1 for sz in sizes]
    NT_PAD = 128

    species2d = species.reshape(N // 128, 128).astype(jnp.int32)
    aev_flat = aev.reshape(N, D)

    d2d, te2d = pl.pallas_call(
        functools.partial(_route_tc_body, [sz // 128 for sz in sizes],
                          NT_PAD),
        out_shape=(jax.ShapeDtypeStruct((N // 128, 128), jnp.int32),
                   jax.ShapeDtypeStruct((1, len(sizes) * NT_PAD), jnp.int32)),
    )(species2d)
    d = d2d.reshape(N)

    mesh2 = plsc.VectorSubcoreMesh(core_axis_name="c", subcore_axis_name="s")

    def make_scatter(atom_base, n_atoms, np_rows):
        return pl.kernel(
            functools.partial(_scatter_rows_body, atom_base, n_atoms),
            out_type=jax.ShapeDtypeStruct((np_rows, D), jnp.float32),
            mesh=mesh2,
            scratch_types=[
                pltpu.VMEM((SUB,), jnp.int32),
                pltpu.VMEM((SUB,), jnp.int32),
                pltpu.VMEM((SUB, D), jnp.float32),
                pltpu.VMEM((SUB, D), jnp.float32),
                pltpu.SemaphoreType.DMA,
                pltpu.SemaphoreType.DMA,
                pltpu.SemaphoreType.DMA,
                pltpu.SemaphoreType.DMA,
                pltpu.SemaphoreType.DMA,
                pltpu.SemaphoreType.DMA,
            ],
        )

    bf = jnp.bfloat16
    W1b = W1.astype(bf)
    W2b = W2.astype(bf)
    W3b = W3.astype(bf)
    W4b = W4.astype(bf)
    B1r = B1.reshape(E, 1, H1)
    B2r = B2.reshape(E, 1, H2)
    B3r = B3.reshape(E, 1, H3)
    B4r = B4.reshape(E, 1, 1)
    b0r = b0.reshape(E, 1, 1)
    b1cr = b1c.reshape(E, 1, 1)

    def run_mlp(piece, nt, np_rows, sorted_aev):
        off = piece * NT_PAD

        def wmap1(i, te_r):
            return (te_r[0, off + jnp.minimum(i, nt - 1)], 0, 0)

        def wmap2(i, te_r):
            return (te_r[0, off + jnp.maximum(i - 1, 0)], 0, 0)

        grid_spec = pltpu.PrefetchScalarGridSpec(
            num_scalar_prefetch=1,
            grid=(nt + 1,),
            in_specs=[
                pl.BlockSpec((T, D),
                             lambda i, te_r: (jnp.minimum(i, nt - 1), 0)),
                pl.BlockSpec((1, H1, D), wmap1),
                pl.BlockSpec((1, 1, H1), wmap1),
                pl.BlockSpec((1, H2, H1), wmap2),
                pl.BlockSpec((1, 1, H2), wmap2),
                pl.BlockSpec((1, H3, H2), wmap2),
                pl.BlockSpec((1, 1, H3), wmap2),
                pl.BlockSpec((1, 1, H3), wmap2),
                pl.BlockSpec((1, 1, 1), wmap2),
                pl.BlockSpec((1, 1, 1), wmap2),
                pl.BlockSpec((1, 1, 1), wmap2),
            ],
            out_specs=pl.BlockSpec(
                (T,), lambda i, te_r: (jnp.maximum(i - 1, 0),)),
            scratch_shapes=[pltpu.VMEM((2, T, H1), jnp.bfloat16)],
        )
        return pl.pallas_call(
            functools.partial(_mlp_body, nt),
            grid_spec=grid_spec,
            out_shape=jax.ShapeDtypeStruct((np_rows,), jnp.float32),
        )(te2d, sorted_aev, W1b, B1r, W2b, B2r, W3b, B3r, W4b, B4r,
          b0r, b1cr)

    def make_gather(atom_base, n_atoms):
        return pl.kernel(
            functools.partial(_gather_out_body, atom_base, n_atoms),
            out_type=jax.ShapeDtypeStruct((n_atoms,), jnp.float32),
            mesh=mesh2,
            scratch_types=[
                pltpu.VMEM((SUB,), jnp.int32),
                pltpu.VMEM((SUB,), jnp.int32),
                pltpu.VMEM((n_atoms // (NC * NS),), jnp.float32),
                pltpu.SemaphoreType.DMA,
                pltpu.SemaphoreType.DMA,
                pltpu.SemaphoreType.DMA,
                pltpu.SemaphoreType.DMA,
            ],
        )

    sorteds = [make_scatter(bases[i], sizes[i], nts[i] * T)(aev_flat, d)
               for i in range(len(sizes))]
    ys = [run_mlp(i, nts[i], nts[i] * T, sorteds[i])
          for i in range(len(sizes))]
    outs = [make_gather(bases[i], sizes[i])(ys[i], d)
            for i in range(len(sizes))]
    return jnp.concatenate(outs).reshape(B, A)
